# Initial kernel scaffold; baseline (speedup 1.0000x reference)
import jax
import jax.numpy as jnp
from jax.experimental import pallas as pl


def kernel(x, edge_index0, edge_index2, W1, b1, W2, b2, fc1_w, fc1_b, fc2_w, fc2_b):
    raise NotImplementedError("write your pallas kernel here")


# speedup vs baseline: 1.0048x; 1.0048x over previous
import functools
import jax
import jax.numpy as jnp
from jax.experimental import pallas as pl
from jax.experimental.pallas import tpu as pltpu


def _cheb(x_nbc, edge_index, Weff, b):
    K = Weff.shape[0]
    N = x_nbc.shape[0]
    row, col = edge_index[0], edge_index[1]
    mask = (row != col).astype(jnp.float32)
    deg = jax.ops.segment_sum(mask, row, num_segments=N)
    dinv = jnp.where(deg > 0, deg ** -0.5, 0.0)
    dinv3 = dinv[:, None, None]

    def prop_scaled(u):
        g = u[col] * mask[:, None, None]
        return jax.ops.segment_sum(g, row, num_segments=N)

    t0 = x_nbc
    u0 = dinv3 * t0
    out = jnp.einsum('nbc,co->nbo', t0, Weff[0])
    t1 = -dinv3 * prop_scaled(u0)
    out = out + jnp.einsum('nbc,co->nbo', t1, Weff[1])
    u1 = dinv3 * t1
    for k in range(2, K):
        t2 = -2.0 * dinv3 * prop_scaled(u1) - t0
        out = out + jnp.einsum('nbc,co->nbo', t2, Weff[k])
        t0, t1 = t1, t2
        u1 = dinv3 * t1
    return out + b


def _fc_kernel(h_ref, w1_ref, b1_ref, w2_ref, b2_ref, out_ref, acc_ref,
               *, nsteps):
    i = pl.program_id(0)

    @pl.when(i == 0)
    def _():
        acc_ref[...] = jnp.zeros_like(acc_ref)

    acc_ref[...] += jnp.dot(h_ref[...], w1_ref[...],
                            preferred_element_type=jnp.float32)

    @pl.when(i == nsteps - 1)
    def _():
        z = jax.nn.relu(acc_ref[...] + b1_ref[...])
        y = jnp.dot(z, w2_ref[...], preferred_element_type=jnp.float32)
        y = y + b2_ref[...]
        m = jnp.max(y, axis=1, keepdims=True)
        e = jnp.exp(y - m)
        lse = jnp.log(jnp.sum(e, axis=1, keepdims=True))
        out_ref[...] = y - m - lse


def _fc_head(hflat, fc1_w, fc1_b, fc2_w, fc2_b):
    B, Kdim = hflat.shape
    D = fc1_w.shape[1]
    C = fc2_w.shape[1]
    BK = 3200
    nsteps = Kdim // BK
    return pl.pallas_call(
        functools.partial(_fc_kernel, nsteps=nsteps),
        grid=(nsteps,),
        in_specs=[
            pl.BlockSpec((B, BK), lambda i: (0, i)),
            pl.BlockSpec((BK, D), lambda i: (i, 0)),
            pl.BlockSpec((1, D), lambda i: (0, 0)),
            pl.BlockSpec((D, C), lambda i: (0, 0)),
            pl.BlockSpec((1, C), lambda i: (0, 0)),
        ],
        out_specs=pl.BlockSpec((B, C), lambda i: (0, 0)),
        out_shape=jax.ShapeDtypeStruct((B, C), jnp.float32),
        scratch_shapes=[pltpu.VMEM((B, D), jnp.float32)],
    )(hflat, fc1_w, fc1_b.reshape(1, D), fc2_w, fc2_b.reshape(1, C))


def kernel(x, edge_index0, edge_index2, W1, b1, W2, b2,
           fc1_w, fc1_b, fc2_w, fc2_b):
    B, N, H = x.shape
    j = jnp.arange(H, dtype=jnp.float32)
    F = jnp.cos((2.0 * jnp.pi / H) * jnp.outer(j, j))
    W1eff = jnp.einsum('hc,kco->kho', F, W1)

    xT = jnp.transpose(x, (1, 0, 2))
    h = _cheb(xT, edge_index0, W1eff, b1)
    h = jax.nn.relu(h)
    N2 = N // 4
    h = h.reshape(N2, 4, B, -1).max(axis=1)
    h = _cheb(h, edge_index2, W2, b2)
    h = jax.nn.relu(h)
    hflat = jnp.transpose(h, (1, 0, 2)).reshape(B, -1)
    return _fc_head(hflat, fc1_w, fc1_b, fc2_w, fc2_b)


# speedup vs baseline: 21.9410x; 21.8358x over previous
import functools
import jax
import jax.numpy as jnp
from jax import lax
from jax.experimental import pallas as pl
from jax.experimental.pallas import tpu as pltpu
from jax.experimental.pallas import tpu_sc as plsc

_NC = 2
_NS = 16
_NW = _NC * _NS


def _vmesh():
    return plsc.VectorSubcoreMesh(core_axis_name="c", subcore_axis_name="s")


def _unpack_rc(rcv):
    col = lax.bitwise_and(rcv, jnp.int32(16383))
    row = lax.shift_right_logical(rcv, jnp.int32(14))
    return row, col


def _sc_deg(rc, NP, EW):
    @functools.partial(
        pl.kernel, mesh=_vmesh(),
        compiler_params=pltpu.CompilerParams(needs_layout_passes=False),
        out_type=jax.ShapeDtypeStruct((_NW, NP), jnp.float32),
        scratch_types=[
            pltpu.VMEM((NP,), jnp.float32),
            pltpu.VMEM((EW,), jnp.int32),
        ],
    )
    def kern(rc_hbm, degp_hbm, deg_v, rce_v):
        wid = lax.axis_index("s") * _NC + lax.axis_index("c")
        pltpu.sync_copy(rc_hbm.at[pl.ds(wid * EW, EW)], rce_v)

        def zero(i, _):
            deg_v[pl.ds(i * 16, 16)] = jnp.zeros((16,), jnp.float32)
            return 0
        lax.fori_loop(0, NP // 16, zero, 0)

        def body(i, _):
            rcv = rce_v[pl.ds(i * 16, 16)]
            row, col = _unpack_rc(rcv)
            v = jnp.where(row != col, jnp.float32(1.0), jnp.float32(0.0))
            plsc.addupdate_scatter(deg_v, [row], v)
            return 0
        lax.fori_loop(0, EW // 16, body, 0)
        pltpu.sync_copy(deg_v, degp_hbm.at[wid])

    return kern(rc)


def _sc_cheb(u_fm, rc, degp, K, CH):
    FP, NP = u_fm.shape
    EP = rc.shape[0]
    NCHUNK = EP // CH
    NPASS = FP // (4 * _NW)

    @functools.partial(
        pl.kernel, mesh=_vmesh(),
        compiler_params=pltpu.CompilerParams(needs_layout_passes=False),
        out_type=jax.ShapeDtypeStruct((K, FP, NP), jnp.float32),
        scratch_types=[
            pltpu.VMEM((4 * NP,), jnp.float32),
            pltpu.VMEM((4 * NP,), jnp.float32),
            pltpu.VMEM((NP,), jnp.float32),
            pltpu.VMEM((NP,), jnp.float32),
            pltpu.VMEM((CH,), jnp.int32),
        ],
    )
    def kern(u_hbm, rc_hbm, degp_hbm, ts_hbm, t0, t1, dinv_v, tmp_v, rcb):
        wid = lax.axis_index("s") * _NC + lax.axis_index("c")

        def zero(i, _):
            dinv_v[pl.ds(i * 16, 16)] = jnp.zeros((16,), jnp.float32)
            return 0
        lax.fori_loop(0, NP // 16, zero, 0)

        def accj(j, _):
            pltpu.sync_copy(degp_hbm.at[j], tmp_v)

            def add(i, _):
                s = pl.ds(i * 16, 16)
                dinv_v[s] = dinv_v[s] + tmp_v[s]
                return 0
            lax.fori_loop(0, NP // 16, add, 0)
            return 0
        lax.fori_loop(0, _NW, accj, 0)

        def rsq(i, _):
            s = pl.ds(i * 16, 16)
            d = dinv_v[s]
            yi = jnp.int32(0x5F3759DF) - lax.shift_right_logical(
                plsc.bitcast(d, jnp.int32), jnp.int32(1))
            y = plsc.bitcast(yi, jnp.float32)
            for _n in range(3):
                y = y * (jnp.float32(1.5) - jnp.float32(0.5) * d * y * y)
            dinv_v[s] = jnp.where(d > jnp.float32(0.5), y, jnp.float32(0.0))
            return 0
        lax.fori_loop(0, NP // 16, rsq, 0)

        def edge_sweep(dst, src, scale):
            def chunk(cix, _):
                pltpu.sync_copy(rc_hbm.at[pl.ds(cix * CH, CH)], rcb)

                def body(i, _):
                    rcv = rcb[pl.ds(i * 16, 16)]
                    row, col = _unpack_rc(rcv)
                    dr = plsc.load_gather(dinv_v, [row])
                    dc = plsc.load_gather(dinv_v, [col])
                    w = jnp.where(row != col, jnp.float32(scale) * dr * dc,
                                  jnp.float32(0.0))
                    for f in range(4):
                        off = jnp.int32(f * NP)
                        g = plsc.load_gather(src, [col + off])
                        plsc.addupdate_scatter(dst, [row + off], w * g)
                    return 0
                lax.fori_loop(0, CH // 16, body, 0)
                return 0
            lax.fori_loop(0, NCHUNK, chunk, 0)

        def fill(dst, negate):
            def body(i, _):
                s = pl.ds(i * 16, 16)
                if negate:
                    dst[s] = -dst[s]
                else:
                    dst[s] = jnp.zeros((16,), jnp.float32)
                return 0
            lax.fori_loop(0, 4 * NP // 16, body, 0)

        def copy_rows(vbuf, hview, to_hbm):
            for f in range(4):
                if to_hbm:
                    pltpu.sync_copy(vbuf.at[pl.ds(f * NP, NP)], hview(f))
                else:
                    pltpu.sync_copy(hview(f), vbuf.at[pl.ds(f * NP, NP)])

        for p in range(NPASS):
            fbase = (wid + p * _NW) * 4
            copy_rows(t0, lambda f: u_hbm.at[fbase + f], to_hbm=False)
            copy_rows(t0, lambda f: ts_hbm.at[0, fbase + f], to_hbm=True)
            fill(t1, negate=False)
            edge_sweep(t1, t0, -1.0)
            copy_rows(t1, lambda f: ts_hbm.at[1, fbase + f], to_hbm=True)
            a, b = t0, t1
            for k in range(2, K):
                fill(a, negate=True)
                edge_sweep(a, b, -2.0)
                copy_rows(a, lambda f, k=k: ts_hbm.at[k, fbase + f],
                          to_hbm=True)
                a, b = b, a

    return kern(u_fm, rc, degp)


def _einsum_body(t_ref, w_ref, b_ref, o_ref, *, pool):
    t = t_ref[...]
    K, _, Cp, NB = t.shape
    y = lax.dot_general(t.reshape(K * Cp, NB), w_ref[...],
                        (((0,), (0,)), ((), ())),
                        preferred_element_type=jnp.float32)
    y = jnp.maximum(y + b_ref[...], 0.0)
    if pool:
        y = y.reshape(NB // 4, 4, y.shape[-1]).max(axis=1)
    o_ref[...] = y.reshape(o_ref.shape)


def _tc_einsum(tstack, Wflat, bias, B, Cp, NB, pool):
    K, FP, NP = tstack.shape
    t4 = tstack.reshape(K, B, Cp, NP)
    O = Wflat.shape[1]
    rows = NB // 4 if pool else NB
    nblk = NP // NB
    return pl.pallas_call(
        functools.partial(_einsum_body, pool=pool),
        grid=(B, nblk),
        in_specs=[
            pl.BlockSpec((K, 1, Cp, NB), lambda b, j: (0, b, 0, j)),
            pl.BlockSpec((K * Cp, O), lambda b, j: (0, 0)),
            pl.BlockSpec((1, O), lambda b, j: (0, 0)),
        ],
        out_specs=pl.BlockSpec((1, rows, O), lambda b, j: (b, j, 0)),
        out_shape=jax.ShapeDtypeStruct((B, NP // 4 if pool else NP, O),
                                       jnp.float32),
    )(t4, Wflat, bias.reshape(1, O))


def _fc_kernel(h_ref, w1_ref, b1_ref, w2_ref, b2_ref, out_ref, acc_ref,
               *, nsteps):
    i = pl.program_id(0)

    @pl.when(i == 0)
    def _():
        acc_ref[...] = jnp.zeros_like(acc_ref)

    acc_ref[...] += jnp.dot(h_ref[...], w1_ref[...],
                            preferred_element_type=jnp.float32)

    @pl.when(i == nsteps - 1)
    def _():
        z = jax.nn.relu(acc_ref[...] + b1_ref[...])
        y = jnp.dot(z, w2_ref[...], preferred_element_type=jnp.float32)
        y = y + b2_ref[...]
        m = jnp.max(y, axis=1, keepdims=True)
        e = jnp.exp(y - m)
        lse = jnp.log(jnp.sum(e, axis=1, keepdims=True))
        out_ref[...] = y - m - lse


def _fc_head(hflat, fc1_w, fc1_b, fc2_w, fc2_b):
    B, Kdim = hflat.shape
    D = fc1_w.shape[1]
    C = fc2_w.shape[1]
    BK = 3200
    nsteps = Kdim // BK
    return pl.pallas_call(
        functools.partial(_fc_kernel, nsteps=nsteps),
        grid=(nsteps,),
        in_specs=[
            pl.BlockSpec((B, BK), lambda i: (0, i)),
            pl.BlockSpec((BK, D), lambda i: (i, 0)),
            pl.BlockSpec((1, D), lambda i: (0, 0)),
            pl.BlockSpec((D, C), lambda i: (0, 0)),
            pl.BlockSpec((1, C), lambda i: (0, 0)),
        ],
        out_specs=pl.BlockSpec((B, C), lambda i: (0, 0)),
        out_shape=jax.ShapeDtypeStruct((B, C), jnp.float32),
        scratch_shapes=[pltpu.VMEM((B, D), jnp.float32)],
    )(hflat, fc1_w, fc1_b.reshape(1, D), fc2_w, fc2_b.reshape(1, C))


def _pack_edges(ei, ep):
    rc = ei[0] * jnp.int32(16384) + ei[1]
    return jnp.pad(rc, (0, ep - rc.shape[0]))


def kernel(x, edge_index0, edge_index2, W1, b1, W2, b2,
           fc1_w, fc1_b, fc2_w, fc2_b):
    B, N, H = x.shape
    K = W1.shape[0]
    G1 = W1.shape[2]
    G2 = W2.shape[2]
    N2 = N // 4
    NP1, NP2 = 10240, 2560
    EP0, EP2 = 163840, 40960
    Cp = 32

    j = jnp.arange(H, dtype=jnp.float32)
    F = jnp.cos((2.0 * jnp.pi / H) * jnp.outer(j, j))
    W1e = jnp.einsum('hc,kco->kho', F, W1)
    Wf1 = jnp.pad(W1e, ((0, 0), (0, Cp - H), (0, 0))).reshape(K * Cp, G1)
    Wf2 = W2.reshape(K * Cp, G2)

    xp = jnp.pad(x, ((0, 0), (0, 0), (0, Cp - H)))
    ufm1 = jnp.pad(xp.transpose(0, 2, 1).reshape(B * Cp, N),
                   ((0, 0), (0, NP1 - N)))

    rc0 = _pack_edges(edge_index0, EP0)
    rc2 = _pack_edges(edge_index2, EP2)

    degp1 = _sc_deg(rc0, NP1, EP0 // _NW)
    ts1 = _sc_cheb(ufm1, rc0, degp1, K, 4096)
    h2d = _tc_einsum(ts1, Wf1, b1, B, Cp, 512, pool=True)

    ufm2 = h2d.transpose(0, 2, 1).reshape(B * G1, NP2)
    degp2 = _sc_deg(rc2, NP2, EP2 // _NW)
    ts2 = _sc_cheb(ufm2, rc2, degp2, K, 4096)
    h3 = _tc_einsum(ts2, Wf2, b2, B, Cp, 512, pool=False)

    hflat = h3[:, :N2, :].reshape(B, N2 * G2)
    return _fc_head(hflat, fc1_w, fc1_b, fc2_w, fc2_b)
